# baseline (speedup 1.0000x reference)
import functools

import jax
import jax.numpy as jnp
from jax import lax
from jax.experimental import pallas as pl
from jax.experimental.pallas import tpu as pltpu
from jax.experimental.pallas import tpu_sc as plsc

E = 160000
EMB = 128
NR = 6
NS = 7
NB = 8

BLK = 1280


def _edge_transform_body(m_ref, rbf_ref, sbf_ref, W_rbf_ref, W_ji_ref, b_ji_ref,
                         W_kj_ref, b_kj_ref, W_sbf_ref,
                         x_ji_ref, x_kj_ref, sbf_p_ref):
    m = m_ref[...]
    rbf_p = rbf_ref[...] @ W_rbf_ref[...]
    x_ji_ref[...] = m @ W_ji_ref[...] + b_ji_ref[...][None, :]
    x_kj_ref[...] = (m @ W_kj_ref[...] + b_kj_ref[...][None, :]) * rbf_p
    sbf_p_ref[...] = sbf_ref[...] @ W_sbf_ref[...]


def _edge_transform(m, rbf, sbf, W_rbf, W_ji, b_ji, W_kj, b_kj, W_sbf):
    nblk = E // BLK
    row = lambda i: (i, 0)
    full = lambda i: (0, 0)
    return pl.pallas_call(
        _edge_transform_body,
        grid=(nblk,),
        in_specs=[
            pl.BlockSpec((BLK, EMB), row),
            pl.BlockSpec((BLK, NR), row),
            pl.BlockSpec((BLK, NR * NS), row),
            pl.BlockSpec((NR, EMB), full),
            pl.BlockSpec((EMB, EMB), full),
            pl.BlockSpec((EMB,), lambda i: (0,)),
            pl.BlockSpec((EMB, EMB), full),
            pl.BlockSpec((EMB,), lambda i: (0,)),
            pl.BlockSpec((NR * NS, NB), full),
        ],
        out_specs=[
            pl.BlockSpec((BLK, EMB), row),
            pl.BlockSpec((BLK, EMB), row),
            pl.BlockSpec((BLK, NB), row),
        ],
        out_shape=[
            jax.ShapeDtypeStruct((E, EMB), jnp.float32),
            jax.ShapeDtypeStruct((E, EMB), jnp.float32),
            jax.ShapeDtypeStruct((E, NB), jnp.float32),
        ],
    )(m, rbf, sbf, W_rbf, W_ji, b_ji, W_kj, b_kj, W_sbf)


def _bilinear_body(xk_ref, sbf_p_ref, Wb_ref, msg_ref):
    xk = xk_ref[...]
    sbf_p = sbf_p_ref[...]
    acc = jnp.zeros((BLK, EMB), jnp.float32)
    for l in range(NB):
        t = jax.lax.dot_general(
            xk, Wb_ref[:, l * EMB:(l + 1) * EMB],
            (((1,), (0,)), ((), ())), preferred_element_type=jnp.float32)
        acc = acc + sbf_p[:, l:l + 1] * t
    msg_ref[...] = acc


def _bilinear(xk, sbf_p, Wb):
    nblk = E // BLK
    return pl.pallas_call(
        _bilinear_body,
        grid=(nblk,),
        in_specs=[
            pl.BlockSpec((BLK, EMB), lambda i: (i, 0)),
            pl.BlockSpec((BLK, NB), lambda i: (i, 0)),
            pl.BlockSpec((EMB, NB * EMB), lambda i: (0, 0)),
        ],
        out_specs=pl.BlockSpec((BLK, EMB), lambda i: (i, 0)),
        out_shape=jax.ShapeDtypeStruct((E, EMB), jnp.float32),
    )(xk, sbf_p, Wb)


def _residual_body(mu_ref, xji_ref, m_ref,
                   w1_ref, b1_ref, w2_ref, b2_ref, wf_ref, bf_ref,
                   a1_ref, ab1_ref, a2_ref, ab2_ref, a3_ref, ab3_ref,
                   a4_ref, ab4_ref, out_ref):
    f32 = jnp.float32
    mm = lambda a, b: jax.lax.dot_general(a, b, (((1,), (0,)), ((), ())),
                                          preferred_element_type=f32)
    h = mu_ref[...] + xji_ref[...]
    h = h + mm(mm(h, w1_ref[...]) + b1_ref[...][None, :], w2_ref[...]) + b2_ref[...][None, :]
    h = mm(h, wf_ref[...]) + bf_ref[...][None, :]
    out = m_ref[...] + h
    out = out + mm(mm(out, a1_ref[...]) + ab1_ref[...][None, :], a2_ref[...]) + ab2_ref[...][None, :]
    out = out + mm(mm(out, a3_ref[...]) + ab3_ref[...][None, :], a4_ref[...]) + ab4_ref[...][None, :]
    out_ref[...] = out


def _residual_stack(m_update, x_ji, m, rb1_W1, rb1_b1, rb1_W2, rb1_b2,
                    W_final, b_final, ra1_W1, ra1_b1, ra1_W2, ra1_b2,
                    ra2_W1, ra2_b1, ra2_W2, ra2_b2):
    nblk = E // BLK
    row = lambda i: (i, 0)
    full = lambda i: (0, 0)
    vec = lambda i: (0,)
    wspec = []
    for _ in range(7):
        wspec += [pl.BlockSpec((EMB, EMB), full), pl.BlockSpec((EMB,), vec)]
    return pl.pallas_call(
        _residual_body,
        grid=(nblk,),
        in_specs=[pl.BlockSpec((BLK, EMB), row)] * 3 + wspec,
        out_specs=pl.BlockSpec((BLK, EMB), row),
        out_shape=jax.ShapeDtypeStruct((E, EMB), jnp.float32),
    )(m_update, x_ji, m, rb1_W1, rb1_b1, rb1_W2, rb1_b2, W_final, b_final,
      ra1_W1, ra1_b1, ra1_W2, ra1_b2, ra2_W1, ra2_b1, ra2_W2, ra2_b2)


_GCHUNK = 200
_NW = 32


def _sc_gather(src, x_kj):
    nchunks_per_w = E // (_GCHUNK * _NW)
    mesh = plsc.VectorSubcoreMesh(core_axis_name="c", subcore_axis_name="s")

    @functools.partial(
        pl.kernel, mesh=mesh,
        compiler_params=pltpu.CompilerParams(needs_layout_passes=False),
        out_type=jax.ShapeDtypeStruct((E, EMB), jnp.float32),
        scratch_types=[
            pltpu.VMEM((_GCHUNK,), jnp.int32),
            pltpu.VMEM((_GCHUNK, EMB), jnp.float32),
            pltpu.SemaphoreType.DMA,
        ],
    )
    def gather_k(src_hbm, xkj_hbm, out_hbm, idx_v, rows_v, sem):
        wid = lax.axis_index("s") * 2 + lax.axis_index("c")

        def body(j, carry):
            base = (wid + _NW * j) * _GCHUNK
            pltpu.sync_copy(src_hbm.at[pl.ds(base, _GCHUNK)], idx_v)
            pltpu.async_copy(xkj_hbm.at[idx_v], rows_v, sem).wait()
            pltpu.sync_copy(rows_v, out_hbm.at[pl.ds(base, _GCHUNK)])
            return carry

        lax.fori_loop(0, nchunks_per_w, body, 0)

    return gather_k(src, x_kj)


_DROWS = 6400
_NRANGE = E // _DROWS
_NPASS = (_NRANGE + 1) // 2
_TRIP = E // 16
_CAP = 10240
_DCHUNK = 128


def _sc_scatter(dst, msg):
    mesh = plsc.VectorSubcoreMesh(core_axis_name="c", subcore_axis_name="s")
    f32 = jnp.float32
    i32 = jnp.int32

    @functools.partial(
        pl.kernel, mesh=mesh,
        compiler_params=pltpu.CompilerParams(needs_layout_passes=False),
        out_type=jax.ShapeDtypeStruct((E, EMB), f32),
        scratch_types=[
            pltpu.VMEM((_TRIP,), i32),
            pltpu.VMEM((_CAP // _DCHUNK, _DCHUNK), i32),
            pltpu.VMEM((_CAP // _DCHUNK, _DCHUNK), i32),
            pltpu.VMEM((_DCHUNK, EMB), f32),
            pltpu.VMEM((16,), i32),
            pltpu.VMEM((200, EMB), f32),
            pltpu.VMEM_SHARED((_DROWS + 8, EMB), f32),
            pltpu.SemaphoreType.DMA,
        ],
    )
    def scatter_k(dst_hbm, msg_hbm, out_hbm,
                  dst_v, widx2, loc2, rows_v, off_v, zeros_v, accum, sem):
        c = lax.axis_index("c")
        s = lax.axis_index("s")
        s_start = s * _TRIP
        garbage = _DROWS

        def zbody(r, _):
            for g in range(EMB // 16):
                zeros_v[r, pl.ds(g * 16, 16)] = jnp.zeros((16,), f32)
            return 0
        lax.fori_loop(0, 200, zbody, 0)
        pltpu.sync_copy(dst_hbm.at[pl.ds(s_start, _TRIP)], dst_v)

        def one_pass(p, _):
            b = 2 * p + c
            lo = b * _DROWS

            @pl.when(b < _NRANGE)
            def _run():
                _pass_body(lo)
            return 0

        def _pass_body(lo):

            for q in range(_DROWS // 16 // 200):
                pltpu.sync_copy(
                    zeros_v, accum.at[pl.ds(s * (_DROWS // 16) + q * 200, 200)])

            @pl.when(s == 0)
            def _():
                pltpu.sync_copy(zeros_v.at[pl.ds(0, 8)],
                                accum.at[pl.ds(_DROWS, 8)])

            def pfbody(rr, _):
                for g in range(_DCHUNK // 16):
                    widx2[rr, pl.ds(g * 16, 16)] = jnp.zeros((16,), i32)
                    loc2[rr, pl.ds(g * 16, 16)] = jnp.full((16,), garbage, i32)
                return 0
            lax.fori_loop(0, _CAP // _DCHUNK, pfbody, 0)

            plsc.subcore_barrier()

            lane = lax.iota(i32, 16)

            off_v[...] = jnp.zeros((16,), i32)

            def fbody(v, _):
                d = dst_v[pl.ds(v * 16, 16)]
                dm = d - lo
                ind = (jnp.minimum(jnp.maximum(dm + 1, 0), 1)
                       * jnp.minimum(jnp.maximum(_DROWS - dm, 0), 1))
                mask = ind > 0
                w = (s_start + v * 16) + lane
                off_vec = off_v[...]
                pos = off_vec * 16 + lane
                prow = pos // _DCHUNK
                pcol = pos - prow * _DCHUNK
                plsc.store_scatter(widx2, [prow, pcol], w, mask=mask)
                plsc.store_scatter(loc2, [prow, pcol], dm, mask=mask)
                off_v[...] = off_vec + ind
                return 0

            lax.fori_loop(0, _TRIP // 16, fbody, 0)
            nmax = jnp.max(off_v[...])
            nchunks = (nmax * 16 + _DCHUNK - 1) // _DCHUNK

            def gbody(j, _):
                pltpu.async_copy(msg_hbm.at[widx2.at[j]], rows_v, sem).wait()
                pltpu.sync_copy(rows_v, accum.at[loc2.at[j]], add=True)
                return 0
            lax.fori_loop(0, nchunks, gbody, 0)

            plsc.subcore_barrier()

            rps = _DROWS // 16
            pltpu.sync_copy(accum.at[pl.ds(s * rps, rps)],
                            out_hbm.at[pl.ds(lo + s * rps, rps)])
            plsc.subcore_barrier()

        lax.fori_loop(0, _NPASS, one_pass, 0)

    return scatter_k(dst, msg)


def kernel(m, rbf, sbf, lg_edge_index, W_rbf, W_sbf, W_ji, b_ji, W_kj, b_kj,
           W_bilin, rb1_W1, rb1_b1, rb1_W2, rb1_b2, W_final, b_final,
           ra1_W1, ra1_b1, ra1_W2, ra1_b2, ra2_W1, ra2_b1, ra2_W2, ra2_b2):
    src = lg_edge_index[0]
    dst = lg_edge_index[1]

    x_ji, x_kj, sbf_p = _edge_transform(m, rbf, sbf, W_rbf, W_ji, b_ji,
                                        W_kj, b_kj, W_sbf)

    xk = _sc_gather(src, x_kj)

    Wb = jnp.reshape(jnp.transpose(W_bilin, (2, 1, 0)), (EMB, NB * EMB))
    msg = _bilinear(xk, sbf_p, Wb)

    m_update = _sc_scatter(dst, msg)

    return _residual_stack(m_update, x_ji, m, rb1_W1, rb1_b1, rb1_W2, rb1_b2,
                           W_final, b_final, ra1_W1, ra1_b1, ra1_W2, ra1_b2,
                           ra2_W1, ra2_b1, ra2_W2, ra2_b2)


# speedup vs baseline: 1.2465x; 1.2465x over previous
import functools

import jax
import jax.numpy as jnp
from jax import lax
from jax.experimental import pallas as pl
from jax.experimental.pallas import tpu as pltpu
from jax.experimental.pallas import tpu_sc as plsc

E = 160000
EMB = 128
NR = 6
NS = 7
NB = 8

BLK = 1280


def _edge_transform_body(m_ref, rbf_ref, sbf_ref, W_rbf_ref, W_ji_ref, b_ji_ref,
                         W_kj_ref, b_kj_ref, W_sbf_ref,
                         x_ji_ref, x_kj_ref, sbf_p_ref):
    m = m_ref[...]
    rbf_p = rbf_ref[...] @ W_rbf_ref[...]
    x_ji_ref[...] = m @ W_ji_ref[...] + b_ji_ref[...][None, :]
    x_kj_ref[...] = (m @ W_kj_ref[...] + b_kj_ref[...][None, :]) * rbf_p
    sbf_p_ref[...] = sbf_ref[...] @ W_sbf_ref[...]


def _edge_transform(m, rbf, sbf, W_rbf, W_ji, b_ji, W_kj, b_kj, W_sbf):
    nblk = E // BLK
    row = lambda i: (i, 0)
    full = lambda i: (0, 0)
    return pl.pallas_call(
        _edge_transform_body,
        grid=(nblk,),
        in_specs=[
            pl.BlockSpec((BLK, EMB), row),
            pl.BlockSpec((BLK, NR), row),
            pl.BlockSpec((BLK, NR * NS), row),
            pl.BlockSpec((NR, EMB), full),
            pl.BlockSpec((EMB, EMB), full),
            pl.BlockSpec((EMB,), lambda i: (0,)),
            pl.BlockSpec((EMB, EMB), full),
            pl.BlockSpec((EMB,), lambda i: (0,)),
            pl.BlockSpec((NR * NS, NB), full),
        ],
        out_specs=[
            pl.BlockSpec((BLK, EMB), row),
            pl.BlockSpec((BLK, EMB), row),
            pl.BlockSpec((BLK, NB), row),
        ],
        out_shape=[
            jax.ShapeDtypeStruct((E, EMB), jnp.float32),
            jax.ShapeDtypeStruct((E, EMB), jnp.float32),
            jax.ShapeDtypeStruct((E, NB), jnp.float32),
        ],
    )(m, rbf, sbf, W_rbf, W_ji, b_ji, W_kj, b_kj, W_sbf)


def _bilinear_body(xk_ref, sbf_p_ref, Wb_ref, msg_ref):
    xk = xk_ref[...]
    sbf_p = sbf_p_ref[...]
    acc = jnp.zeros((BLK, EMB), jnp.float32)
    for l in range(NB):
        t = jax.lax.dot_general(
            xk, Wb_ref[:, l * EMB:(l + 1) * EMB],
            (((1,), (0,)), ((), ())), preferred_element_type=jnp.float32)
        acc = acc + sbf_p[:, l:l + 1] * t
    msg_ref[...] = acc


def _bilinear(xk, sbf_p, Wb):
    nblk = E // BLK
    return pl.pallas_call(
        _bilinear_body,
        grid=(nblk,),
        in_specs=[
            pl.BlockSpec((BLK, EMB), lambda i: (i, 0)),
            pl.BlockSpec((BLK, NB), lambda i: (i, 0)),
            pl.BlockSpec((EMB, NB * EMB), lambda i: (0, 0)),
        ],
        out_specs=pl.BlockSpec((BLK, EMB), lambda i: (i, 0)),
        out_shape=jax.ShapeDtypeStruct((E, EMB), jnp.float32),
    )(xk, sbf_p, Wb)


def _residual_body(mu_ref, xji_ref, m_ref,
                   w1_ref, b1_ref, w2_ref, b2_ref, wf_ref, bf_ref,
                   a1_ref, ab1_ref, a2_ref, ab2_ref, a3_ref, ab3_ref,
                   a4_ref, ab4_ref, out_ref):
    f32 = jnp.float32
    mm = lambda a, b: jax.lax.dot_general(a, b, (((1,), (0,)), ((), ())),
                                          preferred_element_type=f32)
    h = mu_ref[...] + xji_ref[...]
    h = h + mm(mm(h, w1_ref[...]) + b1_ref[...][None, :], w2_ref[...]) + b2_ref[...][None, :]
    h = mm(h, wf_ref[...]) + bf_ref[...][None, :]
    out = m_ref[...] + h
    out = out + mm(mm(out, a1_ref[...]) + ab1_ref[...][None, :], a2_ref[...]) + ab2_ref[...][None, :]
    out = out + mm(mm(out, a3_ref[...]) + ab3_ref[...][None, :], a4_ref[...]) + ab4_ref[...][None, :]
    out_ref[...] = out


def _residual_stack(m_update, x_ji, m, rb1_W1, rb1_b1, rb1_W2, rb1_b2,
                    W_final, b_final, ra1_W1, ra1_b1, ra1_W2, ra1_b2,
                    ra2_W1, ra2_b1, ra2_W2, ra2_b2):
    nblk = E // BLK
    row = lambda i: (i, 0)
    full = lambda i: (0, 0)
    vec = lambda i: (0,)
    wspec = []
    for _ in range(7):
        wspec += [pl.BlockSpec((EMB, EMB), full), pl.BlockSpec((EMB,), vec)]
    return pl.pallas_call(
        _residual_body,
        grid=(nblk,),
        in_specs=[pl.BlockSpec((BLK, EMB), row)] * 3 + wspec,
        out_specs=pl.BlockSpec((BLK, EMB), row),
        out_shape=jax.ShapeDtypeStruct((E, EMB), jnp.float32),
    )(m_update, x_ji, m, rb1_W1, rb1_b1, rb1_W2, rb1_b2, W_final, b_final,
      ra1_W1, ra1_b1, ra1_W2, ra1_b2, ra2_W1, ra2_b1, ra2_W2, ra2_b2)


_GCHUNK = 200
_NW = 32


def _sc_gather(src, x_kj):
    nchunks_per_w = E // (_GCHUNK * _NW)
    mesh = plsc.VectorSubcoreMesh(core_axis_name="c", subcore_axis_name="s")

    @functools.partial(
        pl.kernel, mesh=mesh,
        compiler_params=pltpu.CompilerParams(needs_layout_passes=False),
        out_type=jax.ShapeDtypeStruct((E, EMB), jnp.float32),
        scratch_types=[
            pltpu.VMEM((_GCHUNK,), jnp.int32),
            pltpu.VMEM((_GCHUNK, EMB), jnp.float32),
            pltpu.SemaphoreType.DMA,
        ],
    )
    def gather_k(src_hbm, xkj_hbm, out_hbm, idx_v, rows_v, sem):
        wid = lax.axis_index("s") * 2 + lax.axis_index("c")

        def body(j, carry):
            base = (wid + _NW * j) * _GCHUNK
            pltpu.sync_copy(src_hbm.at[pl.ds(base, _GCHUNK)], idx_v)
            pltpu.async_copy(xkj_hbm.at[idx_v], rows_v, sem).wait()
            pltpu.sync_copy(rows_v, out_hbm.at[pl.ds(base, _GCHUNK)])
            return carry

        lax.fori_loop(0, nchunks_per_w, body, 0)

    return gather_k(src, x_kj)


_DROWS = 10240
_NRANGE = (E + _DROWS - 1) // _DROWS
_NPASS = _NRANGE // 2
_TRIP = E // 16
_CAP = 10112
_DCHUNK = 128


def _sc_scatter(dst, msg):
    mesh = plsc.VectorSubcoreMesh(core_axis_name="c", subcore_axis_name="s")
    f32 = jnp.float32
    i32 = jnp.int32

    @functools.partial(
        pl.kernel, mesh=mesh,
        compiler_params=pltpu.CompilerParams(needs_layout_passes=False),
        out_type=jax.ShapeDtypeStruct((E, EMB), f32),
        scratch_types=[
            pltpu.VMEM((_TRIP,), i32),
            pltpu.VMEM((_CAP // _DCHUNK, _DCHUNK), i32),
            pltpu.VMEM((_CAP // _DCHUNK, _DCHUNK), i32),
            pltpu.VMEM((_DCHUNK, EMB), f32),
            pltpu.VMEM((16,), i32),
            pltpu.VMEM((8, EMB), f32),
            pltpu.VMEM_SHARED((_DROWS + 8, EMB), f32),
            pltpu.SemaphoreType.DMA,
        ],
    )
    def scatter_k(dst_hbm, msg_hbm, out_hbm,
                  dst_v, widx2, loc2, rows_v, off_v, zeros_v, accum, sem):
        c = lax.axis_index("c")
        s = lax.axis_index("s")
        s_start = s * _TRIP
        garbage = _DROWS

        def zbody(r, _):
            for g in range(EMB // 16):
                zeros_v[r, pl.ds(g * 16, 16)] = jnp.zeros((16,), f32)
            return 0
        lax.fori_loop(0, 8, zbody, 0)
        pltpu.sync_copy(dst_hbm.at[pl.ds(s_start, _TRIP)], dst_v)

        def one_pass(p, _):
            b = 2 * p + c
            lo = b * _DROWS

            @pl.when(b < _NRANGE)
            def _run():
                _pass_body(lo)
            return 0

        def _pass_body(lo):

            def azb(q, _):
                pltpu.sync_copy(
                    zeros_v, accum.at[pl.ds(s * (_DROWS // 16) + q * 8, 8)])
                return 0
            lax.fori_loop(0, _DROWS // 16 // 8, azb, 0)

            @pl.when(s == 0)
            def _():
                pltpu.sync_copy(zeros_v.at[pl.ds(0, 8)],
                                accum.at[pl.ds(_DROWS, 8)])

            def pfbody(rr, _):
                for g in range(_DCHUNK // 16):
                    widx2[rr, pl.ds(g * 16, 16)] = jnp.zeros((16,), i32)
                    loc2[rr, pl.ds(g * 16, 16)] = jnp.full((16,), garbage, i32)
                return 0
            lax.fori_loop(0, _CAP // _DCHUNK, pfbody, 0)

            plsc.subcore_barrier()

            lane = lax.iota(i32, 16)

            off_v[...] = jnp.zeros((16,), i32)

            def fbody(v, _):
                d = dst_v[pl.ds(v * 16, 16)]
                dm = d - lo
                ind = (jnp.minimum(jnp.maximum(dm + 1, 0), 1)
                       * jnp.minimum(jnp.maximum(_DROWS - dm, 0), 1))
                mask = ind > 0
                w = (s_start + v * 16) + lane
                off_vec = off_v[...]
                pos = off_vec * 16 + lane
                prow = pos // _DCHUNK
                pcol = pos - prow * _DCHUNK
                plsc.store_scatter(widx2, [prow, pcol], w, mask=mask)
                plsc.store_scatter(loc2, [prow, pcol], dm, mask=mask)
                off_v[...] = off_vec + ind
                return 0

            lax.fori_loop(0, _TRIP // 16, fbody, 0)
            nmax = jnp.max(off_v[...])
            nchunks = (nmax * 16 + _DCHUNK - 1) // _DCHUNK

            def gbody(j, _):
                pltpu.async_copy(msg_hbm.at[widx2.at[j]], rows_v, sem).wait()
                pltpu.sync_copy(rows_v, accum.at[loc2.at[j]], add=True)
                return 0
            lax.fori_loop(0, nchunks, gbody, 0)

            plsc.subcore_barrier()

            rps = _DROWS // 16
            @pl.when(lo + (s + 1) * rps <= E)
            def _dump():
                pltpu.sync_copy(accum.at[pl.ds(s * rps, rps)],
                                out_hbm.at[pl.ds(lo + s * rps, rps)])
            plsc.subcore_barrier()

        lax.fori_loop(0, _NPASS, one_pass, 0)

    return scatter_k(dst, msg)


def kernel(m, rbf, sbf, lg_edge_index, W_rbf, W_sbf, W_ji, b_ji, W_kj, b_kj,
           W_bilin, rb1_W1, rb1_b1, rb1_W2, rb1_b2, W_final, b_final,
           ra1_W1, ra1_b1, ra1_W2, ra1_b2, ra2_W1, ra2_b1, ra2_W2, ra2_b2):
    src = lg_edge_index[0]
    dst = lg_edge_index[1]

    x_ji, x_kj, sbf_p = _edge_transform(m, rbf, sbf, W_rbf, W_ji, b_ji,
                                        W_kj, b_kj, W_sbf)

    xk = _sc_gather(src, x_kj)

    Wb = jnp.reshape(jnp.transpose(W_bilin, (2, 1, 0)), (EMB, NB * EMB))
    msg = _bilinear(xk, sbf_p, Wb)

    m_update = _sc_scatter(dst, msg)

    return _residual_stack(m_update, x_ji, m, rb1_W1, rb1_b1, rb1_W2, rb1_b2,
                           W_final, b_final, ra1_W1, ra1_b1, ra1_W2, ra1_b2,
                           ra2_W1, ra2_b1, ra2_W2, ra2_b2)


# speedup vs baseline: 1.2502x; 1.0030x over previous
import functools

import jax
import jax.numpy as jnp
from jax import lax
from jax.experimental import pallas as pl
from jax.experimental.pallas import tpu as pltpu
from jax.experimental.pallas import tpu_sc as plsc

E = 160000
EMB = 128
NR = 6
NS = 7
NB = 8

BLK = 1280


def _edge_transform_body(m_ref, rbf_ref, sbf_ref, W_rbf_ref, W_ji_ref, b_ji_ref,
                         W_kj_ref, b_kj_ref, W_sbf_ref,
                         x_ji_ref, x_kj_ref, sbf_p_ref):
    m = m_ref[...]
    rbf_p = rbf_ref[...] @ W_rbf_ref[...]
    x_ji_ref[...] = m @ W_ji_ref[...] + b_ji_ref[...][None, :]
    x_kj_ref[...] = (m @ W_kj_ref[...] + b_kj_ref[...][None, :]) * rbf_p
    sbf_p_ref[...] = sbf_ref[...] @ W_sbf_ref[...]


def _edge_transform(m, rbf, sbf, W_rbf, W_ji, b_ji, W_kj, b_kj, W_sbf):
    nblk = E // BLK
    row = lambda i: (i, 0)
    full = lambda i: (0, 0)
    return pl.pallas_call(
        _edge_transform_body,
        grid=(nblk,),
        in_specs=[
            pl.BlockSpec((BLK, EMB), row),
            pl.BlockSpec((BLK, NR), row),
            pl.BlockSpec((BLK, NR * NS), row),
            pl.BlockSpec((NR, EMB), full),
            pl.BlockSpec((EMB, EMB), full),
            pl.BlockSpec((EMB,), lambda i: (0,)),
            pl.BlockSpec((EMB, EMB), full),
            pl.BlockSpec((EMB,), lambda i: (0,)),
            pl.BlockSpec((NR * NS, NB), full),
        ],
        out_specs=[
            pl.BlockSpec((BLK, EMB), row),
            pl.BlockSpec((BLK, EMB), row),
            pl.BlockSpec((BLK, NB), row),
        ],
        out_shape=[
            jax.ShapeDtypeStruct((E, EMB), jnp.float32),
            jax.ShapeDtypeStruct((E, EMB), jnp.float32),
            jax.ShapeDtypeStruct((E, NB), jnp.float32),
        ],
    )(m, rbf, sbf, W_rbf, W_ji, b_ji, W_kj, b_kj, W_sbf)


def _bilinear_body(xk_ref, sbf_p_ref, Wb_ref, msg_ref):
    xk = xk_ref[...]
    sbf_p = sbf_p_ref[...]
    acc = jnp.zeros((BLK, EMB), jnp.float32)
    for l in range(NB):
        t = jax.lax.dot_general(
            xk, Wb_ref[:, l * EMB:(l + 1) * EMB],
            (((1,), (0,)), ((), ())), preferred_element_type=jnp.float32)
        acc = acc + sbf_p[:, l:l + 1] * t
    msg_ref[...] = acc


def _bilinear(xk, sbf_p, Wb):
    nblk = E // BLK
    return pl.pallas_call(
        _bilinear_body,
        grid=(nblk,),
        in_specs=[
            pl.BlockSpec((BLK, EMB), lambda i: (i, 0)),
            pl.BlockSpec((BLK, NB), lambda i: (i, 0)),
            pl.BlockSpec((EMB, NB * EMB), lambda i: (0, 0)),
        ],
        out_specs=pl.BlockSpec((BLK, EMB), lambda i: (i, 0)),
        out_shape=jax.ShapeDtypeStruct((E, EMB), jnp.float32),
    )(xk, sbf_p, Wb)


def _residual_body(mu_ref, xji_ref, m_ref,
                   w1_ref, b1_ref, w2_ref, b2_ref, wf_ref, bf_ref,
                   a1_ref, ab1_ref, a2_ref, ab2_ref, a3_ref, ab3_ref,
                   a4_ref, ab4_ref, out_ref):
    f32 = jnp.float32
    mm = lambda a, b: jax.lax.dot_general(a, b, (((1,), (0,)), ((), ())),
                                          preferred_element_type=f32)
    h = mu_ref[...] + xji_ref[...]
    h = h + mm(mm(h, w1_ref[...]) + b1_ref[...][None, :], w2_ref[...]) + b2_ref[...][None, :]
    h = mm(h, wf_ref[...]) + bf_ref[...][None, :]
    out = m_ref[...] + h
    out = out + mm(mm(out, a1_ref[...]) + ab1_ref[...][None, :], a2_ref[...]) + ab2_ref[...][None, :]
    out = out + mm(mm(out, a3_ref[...]) + ab3_ref[...][None, :], a4_ref[...]) + ab4_ref[...][None, :]
    out_ref[...] = out


def _residual_stack(m_update, x_ji, m, rb1_W1, rb1_b1, rb1_W2, rb1_b2,
                    W_final, b_final, ra1_W1, ra1_b1, ra1_W2, ra1_b2,
                    ra2_W1, ra2_b1, ra2_W2, ra2_b2):
    nblk = E // BLK
    row = lambda i: (i, 0)
    full = lambda i: (0, 0)
    vec = lambda i: (0,)
    wspec = []
    for _ in range(7):
        wspec += [pl.BlockSpec((EMB, EMB), full), pl.BlockSpec((EMB,), vec)]
    return pl.pallas_call(
        _residual_body,
        grid=(nblk,),
        in_specs=[pl.BlockSpec((BLK, EMB), row)] * 3 + wspec,
        out_specs=pl.BlockSpec((BLK, EMB), row),
        out_shape=jax.ShapeDtypeStruct((E, EMB), jnp.float32),
    )(m_update, x_ji, m, rb1_W1, rb1_b1, rb1_W2, rb1_b2, W_final, b_final,
      ra1_W1, ra1_b1, ra1_W2, ra1_b2, ra2_W1, ra2_b1, ra2_W2, ra2_b2)


_GCHUNK = 200
_NW = 32


def _sc_gather(src, x_kj):
    nchunks_per_w = E // (_GCHUNK * _NW)
    mesh = plsc.VectorSubcoreMesh(core_axis_name="c", subcore_axis_name="s")

    @functools.partial(
        pl.kernel, mesh=mesh,
        compiler_params=pltpu.CompilerParams(needs_layout_passes=False),
        out_type=jax.ShapeDtypeStruct((E, EMB), jnp.float32),
        scratch_types=[
            pltpu.VMEM((_GCHUNK,), jnp.int32),
            pltpu.VMEM((_GCHUNK, EMB), jnp.float32),
            pltpu.SemaphoreType.DMA,
        ],
    )
    def gather_k(src_hbm, xkj_hbm, out_hbm, idx_v, rows_v, sem):
        wid = lax.axis_index("s") * 2 + lax.axis_index("c")

        def body(j, carry):
            base = (wid + _NW * j) * _GCHUNK
            pltpu.sync_copy(src_hbm.at[pl.ds(base, _GCHUNK)], idx_v)
            pltpu.async_copy(xkj_hbm.at[idx_v], rows_v, sem).wait()
            pltpu.sync_copy(rows_v, out_hbm.at[pl.ds(base, _GCHUNK)])
            return carry

        lax.fori_loop(0, nchunks_per_w, body, 0)

    return gather_k(src, x_kj)


_DROWS = 10240
_NRANGE = (E + _DROWS - 1) // _DROWS
_NPASS = _NRANGE // 2
_TRIP = E // 16
_CAP = 10112
_DCHUNK = 128


def _sc_scatter(dst, msg):
    mesh = plsc.VectorSubcoreMesh(core_axis_name="c", subcore_axis_name="s")
    f32 = jnp.float32
    i32 = jnp.int32

    @functools.partial(
        pl.kernel, mesh=mesh,
        compiler_params=pltpu.CompilerParams(needs_layout_passes=False),
        out_type=jax.ShapeDtypeStruct((E, EMB), f32),
        scratch_types=[
            pltpu.VMEM((_TRIP,), i32),
            pltpu.VMEM((_CAP // _DCHUNK, _DCHUNK), i32),
            pltpu.VMEM((_CAP // _DCHUNK, _DCHUNK), i32),
            pltpu.VMEM((_DCHUNK, EMB), f32),
            pltpu.VMEM((16,), i32),
            pltpu.VMEM((8, EMB), f32),
            pltpu.VMEM_SHARED((_DROWS + 8, EMB), f32),
            pltpu.SemaphoreType.DMA,
        ],
    )
    def scatter_k(dst_hbm, msg_hbm, out_hbm,
                  dst_v, widx2, loc2, rows_v, off_v, zeros_v, accum, sem):
        c = lax.axis_index("c")
        s = lax.axis_index("s")
        s_start = s * _TRIP
        garbage = _DROWS

        def zbody(r, _):
            for g in range(EMB // 16):
                zeros_v[r, pl.ds(g * 16, 16)] = jnp.zeros((16,), f32)
            return 0
        lax.fori_loop(0, 8, zbody, 0)
        pltpu.sync_copy(dst_hbm.at[pl.ds(s_start, _TRIP)], dst_v)

        def one_pass(p, _):
            b = 2 * p + c
            lo = b * _DROWS

            @pl.when(b < _NRANGE)
            def _run():
                _pass_body(lo)
            return 0

        def _pass_body(lo):

            def azb(q, _):
                pltpu.sync_copy(
                    zeros_v, accum.at[pl.ds(s * (_DROWS // 16) + q * 8, 8)])
                return 0
            lax.fori_loop(0, _DROWS // 16 // 8, azb, 0)

            @pl.when(s == 0)
            def _():
                pltpu.sync_copy(zeros_v.at[pl.ds(0, 8)],
                                accum.at[pl.ds(_DROWS, 8)])

            plsc.subcore_barrier()

            lane = lax.iota(i32, 16)

            off_v[...] = jnp.zeros((16,), i32)

            def fbody(v, _):
                d = dst_v[pl.ds(v * 16, 16)]
                dm = d - lo
                ind = (jnp.minimum(jnp.maximum(dm + 1, 0), 1)
                       * jnp.minimum(jnp.maximum(_DROWS - dm, 0), 1))
                mask = ind > 0
                w = (s_start + v * 16) + lane
                off_vec = off_v[...]
                pos = off_vec * 16 + lane
                prow = pos // _DCHUNK
                pcol = pos - prow * _DCHUNK
                plsc.store_scatter(widx2, [prow, pcol], w, mask=mask)
                plsc.store_scatter(loc2, [prow, pcol], dm, mask=mask)
                off_v[...] = off_vec + ind
                return 0

            lax.fori_loop(0, _TRIP // 16, fbody, 0)
            nmax = jnp.max(off_v[...])
            nchunks = (nmax * 16 + _DCHUNK - 1) // _DCHUNK

            zero16 = jnp.zeros((16,), i32)
            garb16 = jnp.full((16,), garbage, i32)

            def cbody(rr, _):
                rrv = zero16 + rr
                offv = off_v[...]
                for g in range(_DCHUNK // 16):
                    empty = (rrv * 8 + g) >= offv
                    col = g * 16 + lane
                    plsc.store_scatter(widx2, [rrv, col], zero16, mask=empty)
                    plsc.store_scatter(loc2, [rrv, col], garb16, mask=empty)
                return 0
            lax.fori_loop(0, nchunks, cbody, 0)

            def gbody(j, _):
                pltpu.async_copy(msg_hbm.at[widx2.at[j]], rows_v, sem).wait()
                pltpu.sync_copy(rows_v, accum.at[loc2.at[j]], add=True)
                return 0
            lax.fori_loop(0, nchunks, gbody, 0)

            plsc.subcore_barrier()

            rps = _DROWS // 16
            @pl.when(lo + (s + 1) * rps <= E)
            def _dump():
                pltpu.sync_copy(accum.at[pl.ds(s * rps, rps)],
                                out_hbm.at[pl.ds(lo + s * rps, rps)])
            plsc.subcore_barrier()

        lax.fori_loop(0, _NPASS, one_pass, 0)

    return scatter_k(dst, msg)


def kernel(m, rbf, sbf, lg_edge_index, W_rbf, W_sbf, W_ji, b_ji, W_kj, b_kj,
           W_bilin, rb1_W1, rb1_b1, rb1_W2, rb1_b2, W_final, b_final,
           ra1_W1, ra1_b1, ra1_W2, ra1_b2, ra2_W1, ra2_b1, ra2_W2, ra2_b2):
    src = lg_edge_index[0]
    dst = lg_edge_index[1]

    x_ji, x_kj, sbf_p = _edge_transform(m, rbf, sbf, W_rbf, W_ji, b_ji,
                                        W_kj, b_kj, W_sbf)

    xk = _sc_gather(src, x_kj)

    Wb = jnp.reshape(jnp.transpose(W_bilin, (2, 1, 0)), (EMB, NB * EMB))
    msg = _bilinear(xk, sbf_p, Wb)

    m_update = _sc_scatter(dst, msg)

    return _residual_stack(m_update, x_ji, m, rb1_W1, rb1_b1, rb1_W2, rb1_b2,
                           W_final, b_final, ra1_W1, ra1_b1, ra1_W2, ra1_b2,
                           ra2_W1, ra2_b1, ra2_W2, ra2_b2)
